# Initial kernel scaffold; baseline (speedup 1.0000x reference)
import jax
import jax.numpy as jnp
from jax.experimental import pallas as pl


def kernel(W):
    raise NotImplementedError("write your pallas kernel here")


# speedup vs baseline: 3.4765x; 3.4765x over previous
import functools

import jax
import jax.numpy as jnp
from jax.experimental import pallas as pl

_N = 4096
_D = 128
_BM = 512
_BN = 512


def _norm_body(w_ref, out_ref):
    w = w_ref[...]
    n = jnp.sqrt(jnp.sum(w * w, axis=1, keepdims=True))
    out_ref[...] = w / jnp.maximum(n, 1e-8)


def _main_body(wi_ref, wj_ref, a_ref, attr_ref, idx_ref, *, bm, bn):
    i = pl.program_id(0)
    j = pl.program_id(1)
    a = jnp.dot(wi_ref[...], wj_ref[...].T, preferred_element_type=jnp.float32)
    row = jax.lax.broadcasted_iota(jnp.int32, (bm, bn), 0) + i * bm
    col = jax.lax.broadcasted_iota(jnp.int32, (bm, bn), 1) + j * bn
    a = jnp.where(row == col, 0.0, jnp.maximum(a, 0.0))
    a_ref[...] = a
    attr_ref[...] = a
    idx_ref[0] = row
    idx_ref[1] = col


def kernel(W):
    Wn = pl.pallas_call(
        _norm_body,
        out_shape=jax.ShapeDtypeStruct((_N, _D), jnp.float32),
    )(W)

    grid = (_N // _BM, _N // _BN)
    a, attr, idx = pl.pallas_call(
        functools.partial(_main_body, bm=_BM, bn=_BN),
        grid=grid,
        in_specs=[
            pl.BlockSpec((_BM, _D), lambda i, j: (i, 0)),
            pl.BlockSpec((_BN, _D), lambda i, j: (j, 0)),
        ],
        out_specs=[
            pl.BlockSpec((_BM, _BN), lambda i, j: (i, j)),
            pl.BlockSpec((_BM, _BN), lambda i, j: (i, j)),
            pl.BlockSpec((2, _BM, _BN), lambda i, j: (0, i, j)),
        ],
        out_shape=[
            jax.ShapeDtypeStruct((_N, _N), jnp.float32),
            jax.ShapeDtypeStruct((_N, _N), jnp.float32),
            jax.ShapeDtypeStruct((2, _N, _N), jnp.int32),
        ],
    )(Wn, Wn)

    edge_indices = idx.reshape(2, _N * _N)
    edge_attr = attr.reshape(_N * _N)
    return (edge_indices, edge_attr, a)


# speedup vs baseline: 3.6417x; 1.0475x over previous
import functools

import jax
import jax.numpy as jnp
from jax.experimental import pallas as pl

_N = 4096
_D = 128
_BM = 1024
_BN = 1024


def _norm_body(w_ref, out_ref):
    w = w_ref[...]
    n = jnp.sqrt(jnp.sum(w * w, axis=1, keepdims=True))
    out_ref[...] = w / jnp.maximum(n, 1e-8)


def _main_body(wi_ref, wj_ref, a_ref, attr_ref, idx_ref, *, bm, bn):
    i = pl.program_id(0)
    j = pl.program_id(1)
    a = jnp.dot(wi_ref[...], wj_ref[...].T, preferred_element_type=jnp.float32)
    row = jax.lax.broadcasted_iota(jnp.int32, (bm, bn), 0) + i * bm
    col = jax.lax.broadcasted_iota(jnp.int32, (bm, bn), 1) + j * bn
    a = jnp.where(row == col, 0.0, jnp.maximum(a, 0.0))
    a_ref[...] = a
    attr_ref[...] = a
    idx_ref[0] = row
    idx_ref[1] = col


def kernel(W):
    Wn = pl.pallas_call(
        _norm_body,
        out_shape=jax.ShapeDtypeStruct((_N, _D), jnp.float32),
    )(W)

    grid = (_N // _BM, _N // _BN)
    a, attr, idx = pl.pallas_call(
        functools.partial(_main_body, bm=_BM, bn=_BN),
        grid=grid,
        in_specs=[
            pl.BlockSpec((_BM, _D), lambda i, j: (i, 0)),
            pl.BlockSpec((_BN, _D), lambda i, j: (j, 0)),
        ],
        out_specs=[
            pl.BlockSpec((_BM, _BN), lambda i, j: (i, j)),
            pl.BlockSpec((_BM, _BN), lambda i, j: (i, j)),
            pl.BlockSpec((2, _BM, _BN), lambda i, j: (0, i, j)),
        ],
        out_shape=[
            jax.ShapeDtypeStruct((_N, _N), jnp.float32),
            jax.ShapeDtypeStruct((_N, _N), jnp.float32),
            jax.ShapeDtypeStruct((2, _N, _N), jnp.int32),
        ],
    )(Wn, Wn)

    edge_indices = idx.reshape(2, _N * _N)
    edge_attr = attr.reshape(_N * _N)
    return (edge_indices, edge_attr, a)


# speedup vs baseline: 3.6585x; 1.0046x over previous
import functools

import jax
import jax.numpy as jnp
from jax.experimental import pallas as pl

_N = 4096
_D = 128
_BM = 256
_BN = 4096


def _norm_body(w_ref, out_ref):
    w = w_ref[...]
    n = jnp.sqrt(jnp.sum(w * w, axis=1, keepdims=True))
    out_ref[...] = w / jnp.maximum(n, 1e-8)


def _main_body(wi_ref, wj_ref, a_ref, attr_ref, idx_ref, *, bm, bn):
    i = pl.program_id(0)
    j = pl.program_id(1)
    a = jnp.dot(wi_ref[...], wj_ref[...].T, preferred_element_type=jnp.float32)
    row = jax.lax.broadcasted_iota(jnp.int32, (bm, bn), 0) + i * bm
    col = jax.lax.broadcasted_iota(jnp.int32, (bm, bn), 1) + j * bn
    a = jnp.where(row == col, 0.0, jnp.maximum(a, 0.0))
    a_ref[...] = a
    attr_ref[...] = a
    idx_ref[0] = row
    idx_ref[1] = col


def kernel(W):
    Wn = pl.pallas_call(
        _norm_body,
        out_shape=jax.ShapeDtypeStruct((_N, _D), jnp.float32),
    )(W)

    grid = (_N // _BM, _N // _BN)
    a, attr, idx = pl.pallas_call(
        functools.partial(_main_body, bm=_BM, bn=_BN),
        grid=grid,
        in_specs=[
            pl.BlockSpec((_BM, _D), lambda i, j: (i, 0)),
            pl.BlockSpec((_BN, _D), lambda i, j: (j, 0)),
        ],
        out_specs=[
            pl.BlockSpec((_BM, _BN), lambda i, j: (i, j)),
            pl.BlockSpec((_BM, _BN), lambda i, j: (i, j)),
            pl.BlockSpec((2, _BM, _BN), lambda i, j: (0, i, j)),
        ],
        out_shape=[
            jax.ShapeDtypeStruct((_N, _N), jnp.float32),
            jax.ShapeDtypeStruct((_N, _N), jnp.float32),
            jax.ShapeDtypeStruct((2, _N, _N), jnp.int32),
        ],
    )(Wn, Wn)

    edge_indices = idx.reshape(2, _N * _N)
    edge_attr = attr.reshape(_N * _N)
    return (edge_indices, edge_attr, a)


# speedup vs baseline: 4.5617x; 1.2469x over previous
import functools

import jax
import jax.numpy as jnp
from jax.experimental import pallas as pl

_N = 4096
_D = 128
_F = _N * _N // 128
_BM = 256
_BL = 8192


def _norm_body(w_ref, out_ref):
    w = w_ref[...]
    n = jnp.sqrt(jnp.sum(w * w, axis=1, keepdims=True))
    out_ref[...] = w / jnp.maximum(n, 1e-8)


def _main_body(wi_ref, wall_ref, a_ref, attr_ref, *, bm):
    i = pl.program_id(0)
    a = jnp.dot(wi_ref[...], wall_ref[...].T, preferred_element_type=jnp.float32)
    row = jax.lax.broadcasted_iota(jnp.int32, (bm, _N), 0) + i * bm
    col = jax.lax.broadcasted_iota(jnp.int32, (bm, _N), 1)
    a = jnp.where(row == col, 0.0, jnp.maximum(a, 0.0))
    a_ref[...] = a
    attr_ref[...] = a.reshape(bm * 32, 128)


def _idx_body(idx_ref, *, bl):
    i = pl.program_id(0)
    s = jax.lax.broadcasted_iota(jnp.int32, (bl, 128), 0) + i * bl
    l = jax.lax.broadcasted_iota(jnp.int32, (bl, 128), 1)
    idx_ref[0] = s >> 5
    idx_ref[1] = ((s & 31) << 7) | l


def kernel(W):
    Wn = pl.pallas_call(
        _norm_body,
        out_shape=jax.ShapeDtypeStruct((_N, _D), jnp.float32),
    )(W)

    a, attr = pl.pallas_call(
        functools.partial(_main_body, bm=_BM),
        grid=(_N // _BM,),
        in_specs=[
            pl.BlockSpec((_BM, _D), lambda i: (i, 0)),
            pl.BlockSpec((_N, _D), lambda i: (0, 0)),
        ],
        out_specs=[
            pl.BlockSpec((_BM, _N), lambda i: (i, 0)),
            pl.BlockSpec((_BM * 32, 128), lambda i: (i, 0)),
        ],
        out_shape=[
            jax.ShapeDtypeStruct((_N, _N), jnp.float32),
            jax.ShapeDtypeStruct((_F, 128), jnp.float32),
        ],
    )(Wn, Wn)

    idx = pl.pallas_call(
        functools.partial(_idx_body, bl=_BL),
        grid=(_F // _BL,),
        out_specs=pl.BlockSpec((2, _BL, 128), lambda i: (0, i, 0)),
        out_shape=jax.ShapeDtypeStruct((2, _F, 128), jnp.int32),
    )()

    edge_indices = idx.reshape(2, _N * _N)
    edge_attr = attr.reshape(_N * _N)
    return (edge_indices, edge_attr, a)
